# baseline (speedup 1.0000x reference)
import functools

import jax
import jax.numpy as jnp
from jax import lax
from jax.experimental import pallas as pl
from jax.experimental.pallas import tpu as pltpu
from jax.experimental.pallas import tpu_sc as plsc

N = 100000
D = 128
G = 64
SCALE = float(1562.5) ** (-0.5)

NC = 2
L = 16
NS = 16
NW = NC * NS
GROUP = 128
BLOCK = 256
GPB = BLOCK // GROUP
NBLK = N // BLOCK
TAILBLK = NBLK
TAIL_ROWS = N - NBLK * BLOCK
TAIL_REM = TAIL_ROWS - GROUP
BPW = -(-(NBLK + 1) // NW)
NPAIR = (BPW + 1) // 2
IDROWS = -(-N // GROUP) + 1
SEGS_PER_TILE = G // NS


@functools.partial(
    pl.kernel,
    out_type=jax.ShapeDtypeStruct((NC, G, D), jnp.float32),
    mesh=plsc.VectorSubcoreMesh(core_axis_name="c", subcore_axis_name="s"),
    compiler_params=pltpu.CompilerParams(use_tc_tiling_on_sc=False),
    scratch_types=[
        pltpu.VMEM((2, BLOCK, D), jnp.float32),
        pltpu.VMEM((2, GPB, GROUP), jnp.int32),
        pltpu.VMEM((TAIL_ROWS, D), jnp.float32),
        pltpu.VMEM((GROUP,), jnp.int32),
        pltpu.VMEM((TAIL_REM,), jnp.int32),
        pltpu.VMEM((SEGS_PER_TILE, D), jnp.float32),
        pltpu.VMEM((G, D), jnp.float32),
        pltpu.VMEM((NS, SEGS_PER_TILE, D), jnp.float32),
        pltpu.VMEM_SHARED((NS, G, D), jnp.float32),
        pltpu.SemaphoreType.DMA,
        pltpu.SemaphoreType.DMA,
        pltpu.SemaphoreType.DMA,
        pltpu.SemaphoreType.DMA,
        pltpu.SemaphoreType.DMA,
        pltpu.SemaphoreType.DMA,
    ],
)
def _sc_segment_sum(nodes_ref, ids_ref, zeros_ref, part_ref,
                    nbuf, ibuf, tnbuf, tidx_a, tidx_b, outbuf, acc, rbuf,
                    acc_all, nsem0, nsem1, isem0, isem1, tnsem, tisem):
    c = lax.axis_index("c")
    s = lax.axis_index("s")
    w = s * NC + c
    nsems = (nsem0, nsem1)
    isems = (isem0, isem1)

    pltpu.sync_copy(zeros_ref, acc)

    def node_copy(b, slot):
        return pltpu.make_async_copy(
            nodes_ref.at[pl.ds(b * BLOCK, BLOCK)], nbuf.at[slot], nsems[slot])

    def id_copy(b, slot):
        return pltpu.make_async_copy(
            ids_ref.at[pl.ds(b * GPB, GPB)], ibuf.at[slot], isems[slot])

    def tail_copies():
        r0 = NBLK * BLOCK
        return (
            pltpu.make_async_copy(
                nodes_ref.at[pl.ds(r0, TAIL_ROWS)], tnbuf, tnsem),
            pltpu.make_async_copy(ids_ref.at[NBLK * GPB], tidx_a, tisem),
            pltpu.make_async_copy(
                ids_ref.at[NBLK * GPB + 1, pl.ds(0, TAIL_REM)], tidx_b, tisem),
        )

    def start_load(k, slot):
        b = w * BPW + k
        in_range = k < BPW

        @pl.when(in_range & (b < NBLK))
        def _():
            node_copy(b, slot).start()
            id_copy(b, slot).start()

        @pl.when(in_range & (b == TAILBLK))
        def _():
            for cp in tail_copies():
                cp.start()

    def consume(k, slot):
        b = w * BPW + k
        in_range = k < BPW

        @pl.when(in_range & (b < NBLK))
        def _():
            node_copy(b, slot).wait()
            id_copy(b, slot).wait()

            for g in range(GPB):
                def chunk(m, carry, g=g):
                    ids16 = ibuf[slot, g, pl.ds(m * L, L)]
                    for j in range(L):
                        seg = ids16[j]
                        r = g * GROUP + m * L + j
                        for l in range(D // L):
                            plsc.addupdate(
                                acc.at[seg, pl.ds(l * L, L)],
                                nbuf[slot, r, pl.ds(l * L, L)])
                    return carry

                lax.fori_loop(0, GROUP // L, chunk, 0)

        @pl.when(in_range & (b == TAILBLK))
        def _():
            for cp in tail_copies():
                cp.wait()

            def tchunk(m, carry):
                ids16 = tidx_a[pl.ds(m * L, L)]
                for j in range(L):
                    seg = ids16[j]
                    r = m * L + j
                    for l in range(D // L):
                        plsc.addupdate(
                            acc.at[seg, pl.ds(l * L, L)],
                            tnbuf[r, pl.ds(l * L, L)])
                return carry

            lax.fori_loop(0, GROUP // L, tchunk, 0)

            def tchunk2(m, carry):
                ids16 = tidx_b[pl.ds(m * L, L)]
                for j in range(L):
                    seg = ids16[j]
                    r = GROUP + m * L + j
                    for l in range(D // L):
                        plsc.addupdate(
                            acc.at[seg, pl.ds(l * L, L)],
                            tnbuf[r, pl.ds(l * L, L)])
                return carry

            lax.fori_loop(0, TAIL_REM // L, tchunk2, 0)

    start_load(0, 0)
    start_load(1, 1)

    def pair(i, carry):
        k = i * 2
        consume(k, 0)
        start_load(k + 2, 0)
        consume(k + 1, 1)
        start_load(k + 3, 1)
        return carry

    lax.fori_loop(0, NPAIR, pair, 0)

    pltpu.sync_copy(acc, acc_all.at[s])
    plsc.subcore_barrier()

    seg0 = s * SEGS_PER_TILE
    pltpu.sync_copy(acc_all.at[:, pl.ds(seg0, SEGS_PER_TILE)], rbuf)
    for i in range(SEGS_PER_TILE):
        for l in range(D // L):
            v = rbuf[0, i, pl.ds(l * L, L)]
            for t in range(1, NS):
                v = v + rbuf[t, i, pl.ds(l * L, L)]
            outbuf[i, pl.ds(l * L, L)] = v
    pltpu.sync_copy(outbuf, part_ref.at[c, pl.ds(seg0, SEGS_PER_TILE)])


def _combine_body(p_ref, o_ref):
    o_ref[...] = (p_ref[0] + p_ref[1]) * SCALE


def kernel(nodes, segment_ids, num_segments):
    ids = segment_ids.astype(jnp.int32)
    ids = jnp.pad(ids, (0, IDROWS * GROUP - N)).reshape(IDROWS, GROUP)
    zeros = jnp.zeros((G, D), jnp.float32)
    partials = _sc_segment_sum(nodes, ids, zeros)
    return pl.pallas_call(
        _combine_body,
        out_shape=jax.ShapeDtypeStruct((G, D), jnp.float32),
    )(partials)


# speedup vs baseline: 2.1311x; 2.1311x over previous
import functools

import jax
import jax.numpy as jnp
from jax import lax
from jax.experimental import pallas as pl
from jax.experimental.pallas import tpu as pltpu
from jax.experimental.pallas import tpu_sc as plsc

N = 100000
D = 128
G = 64
SCALE = float(1562.5) ** (-0.5)

NC = 2
NS = 16
NW = NC * NS
L = 16
GROUP = 128
BLOCK = 256
GPB = BLOCK // GROUP
NBLK = N // BLOCK
TAILBLK = NBLK
TAIL_ROWS = N - NBLK * BLOCK
TAIL_REM = TAIL_ROWS - GROUP
BPW = -(-(NBLK + 1) // NW)
NPAIR = (BPW + 1) // 2
SEGS_PER_TILE = G // NS


@functools.partial(
    pl.kernel,
    out_type=jax.ShapeDtypeStruct((NC, G, D), jnp.float32),
    mesh=plsc.VectorSubcoreMesh(core_axis_name="c", subcore_axis_name="s"),
    compiler_params=pltpu.CompilerParams(use_tc_tiling_on_sc=False),
    scratch_types=[
        pltpu.VMEM((2, BLOCK, D), jnp.float32),
        pltpu.VMEM((2, GPB, GROUP), jnp.int32),
        pltpu.VMEM((TAIL_ROWS, D), jnp.float32),
        pltpu.VMEM((GROUP,), jnp.int32),
        pltpu.VMEM((TAIL_REM,), jnp.int32),
        pltpu.VMEM((SEGS_PER_TILE, D), jnp.float32),
        pltpu.VMEM_SHARED((G, D), jnp.float32),
        pltpu.SemaphoreType.DMA,
        pltpu.SemaphoreType.DMA,
        pltpu.SemaphoreType.DMA,
        pltpu.SemaphoreType.DMA,
        pltpu.SemaphoreType.DMA,
        pltpu.SemaphoreType.DMA,
        pltpu.SemaphoreType.DMA,
        pltpu.SemaphoreType.DMA,
    ],
)
def _sc_segment_sum(nodes_ref, ids_ref, part_ref,
                    nbuf, ibuf, tnbuf, tidx_a, tidx_b, outbuf, acc,
                    nsem0, nsem1, isem0, isem1, ssem0, ssem1, tnsem, tisem):
    c = lax.axis_index("c")
    s = lax.axis_index("s")
    w = s * NC + c
    nsems = (nsem0, nsem1)
    isems = (isem0, isem1)
    ssems = (ssem0, ssem1)

    zvec = jnp.zeros((L,), jnp.float32)
    for i in range(SEGS_PER_TILE):
        for l in range(D // L):
            outbuf[i, pl.ds(l * L, L)] = zvec
    pltpu.sync_copy(outbuf, acc.at[pl.ds(s * SEGS_PER_TILE, SEGS_PER_TILE)])
    plsc.subcore_barrier()

    def node_copy(b, slot):
        return pltpu.make_async_copy(
            nodes_ref.at[pl.ds(b * BLOCK, BLOCK)], nbuf.at[slot], nsems[slot])

    def id_copies(b, slot):
        return tuple(
            pltpu.make_async_copy(
                ids_ref.at[pl.ds(b * BLOCK + g * GROUP, GROUP)],
                ibuf.at[slot, g], isems[slot])
            for g in range(GPB))

    def scatter_copies(slot):
        return tuple(
            pltpu.make_async_copy(
                nbuf.at[slot, pl.ds(g * GROUP, GROUP)],
                acc.at[ibuf.at[slot, g]], ssems[slot])
            for g in range(GPB))

    def tail_copies():
        r0 = NBLK * BLOCK
        return (
            pltpu.make_async_copy(
                nodes_ref.at[pl.ds(r0, TAIL_ROWS)], tnbuf, tnsem),
            pltpu.make_async_copy(
                ids_ref.at[pl.ds(r0, GROUP)], tidx_a, tisem),
            pltpu.make_async_copy(
                ids_ref.at[pl.ds(r0 + GROUP, TAIL_REM)], tidx_b, tisem),
        )

    def start_load(k, slot):
        b = w * BPW + k
        in_range = k < BPW

        @pl.when(in_range & (b < NBLK))
        def _():
            node_copy(b, slot).start()
            for cp in id_copies(b, slot):
                cp.start()

        @pl.when(in_range & (b == TAILBLK))
        def _():
            for cp in tail_copies():
                cp.start()

    def consume_start(k, slot):
        b = w * BPW + k
        in_range = k < BPW

        @pl.when(in_range & (b < NBLK))
        def _():
            node_copy(b, slot).wait()
            for cp in id_copies(b, slot):
                cp.wait()
            for cp in scatter_copies(slot):
                cp.start(add=True)

        @pl.when(in_range & (b == TAILBLK))
        def _():
            for cp in tail_copies():
                cp.wait()
            pltpu.sync_copy(
                tnbuf.at[pl.ds(0, GROUP)], acc.at[tidx_a], add=True)
            pltpu.sync_copy(
                tnbuf.at[pl.ds(GROUP, TAIL_REM)], acc.at[tidx_b], add=True)

    def scatter_drain(k, slot):
        b = w * BPW + k
        in_range = k < BPW

        @pl.when(in_range & (b < NBLK))
        def _():
            for cp in scatter_copies(slot):
                cp.wait()

    start_load(0, 0)
    start_load(1, 1)

    def pair(i, carry):
        k = i * 2
        consume_start(k, 0)
        consume_start(k + 1, 1)
        scatter_drain(k, 0)
        start_load(k + 2, 0)
        scatter_drain(k + 1, 1)
        start_load(k + 3, 1)
        return carry

    lax.fori_loop(0, NPAIR, pair, 0)

    plsc.subcore_barrier()

    seg0 = s * SEGS_PER_TILE
    pltpu.sync_copy(acc.at[pl.ds(seg0, SEGS_PER_TILE)], outbuf)
    pltpu.sync_copy(outbuf, part_ref.at[c, pl.ds(seg0, SEGS_PER_TILE)])


def _combine_body(p_ref, o_ref):
    o_ref[...] = (p_ref[0] + p_ref[1]) * SCALE


def kernel(nodes, segment_ids, num_segments):
    ids = segment_ids.astype(jnp.int32)
    partials = _sc_segment_sum(nodes, ids)
    return pl.pallas_call(
        _combine_body,
        out_shape=jax.ShapeDtypeStruct((G, D), jnp.float32),
    )(partials)


# speedup vs baseline: 2.5762x; 1.2088x over previous
import functools

import jax
import jax.numpy as jnp
from jax import lax
from jax.experimental import pallas as pl
from jax.experimental.pallas import tpu as pltpu
from jax.experimental.pallas import tpu_sc as plsc

N = 100000
D = 128
G = 64
SCALE = float(1562.5) ** (-0.5)

NC = 2
NS = 16
NW = NC * NS
L = 16
GROUP = 128
BLOCK = 256
GPB = BLOCK // GROUP
NBLK = N // BLOCK
TAILBLK = NBLK
TAIL_ROWS = N - NBLK * BLOCK
TAIL_REM = TAIL_ROWS - GROUP
BPW = -(-(NBLK + 1) // NW)
NPAIR = (BPW + 1) // 2
SEGS_PER_TILE = G // NS


@functools.partial(
    pl.kernel,
    out_type=jax.ShapeDtypeStruct((NC, G, D), jnp.float32),
    mesh=plsc.VectorSubcoreMesh(core_axis_name="c", subcore_axis_name="s"),
    compiler_params=pltpu.CompilerParams(use_tc_tiling_on_sc=False),
    scratch_types=[
        pltpu.VMEM((2, BLOCK, D), jnp.float32),
        pltpu.VMEM((2, GPB, GROUP), jnp.int32),
        pltpu.VMEM((TAIL_ROWS, D), jnp.float32),
        pltpu.VMEM((GROUP,), jnp.int32),
        pltpu.VMEM((TAIL_REM,), jnp.int32),
        pltpu.VMEM((SEGS_PER_TILE, D), jnp.float32),
        pltpu.VMEM_SHARED((G, D), jnp.float32),
        pltpu.SemaphoreType.DMA,
        pltpu.SemaphoreType.DMA,
        pltpu.SemaphoreType.DMA,
        pltpu.SemaphoreType.DMA,
        pltpu.SemaphoreType.DMA,
        pltpu.SemaphoreType.DMA,
    ],
)
def _sc_segment_sum(nodes_ref, ids_ref, part_ref,
                    nbuf, ibuf, tnbuf, tidx_a, tidx_b, outbuf, acc,
                    nsem0, nsem1, isem0, isem1, tnsem, tisem):
    c = lax.axis_index("c")
    s = lax.axis_index("s")
    w = s * NC + c
    nsems = (nsem0, nsem1)
    isems = (isem0, isem1)

    zvec = jnp.zeros((L,), jnp.float32)
    for i in range(SEGS_PER_TILE):
        for l in range(D // L):
            outbuf[i, pl.ds(l * L, L)] = zvec
    pltpu.sync_copy(outbuf, acc.at[pl.ds(s * SEGS_PER_TILE, SEGS_PER_TILE)])
    plsc.subcore_barrier()

    def node_copy(b, slot):
        return pltpu.make_async_copy(
            nodes_ref.at[pl.ds(b * BLOCK, BLOCK)], nbuf.at[slot], nsems[slot])

    def id_copies(b, slot):
        return tuple(
            pltpu.make_async_copy(
                ids_ref.at[pl.ds(b * BLOCK + g * GROUP, GROUP)],
                ibuf.at[slot, g], isems[slot])
            for g in range(GPB))

    def tail_copies():
        r0 = NBLK * BLOCK
        return (
            pltpu.make_async_copy(
                nodes_ref.at[pl.ds(r0, TAIL_ROWS)], tnbuf, tnsem),
            pltpu.make_async_copy(
                ids_ref.at[pl.ds(r0, GROUP)], tidx_a, tisem),
            pltpu.make_async_copy(
                ids_ref.at[pl.ds(r0 + GROUP, TAIL_REM)], tidx_b, tisem),
        )

    def start_load(k, slot):
        b = w * BPW + k
        in_range = k < BPW

        @pl.when(in_range & (b < NBLK))
        def _():
            node_copy(b, slot).start()
            for cp in id_copies(b, slot):
                cp.start()

        @pl.when(in_range & (b == TAILBLK))
        def _():
            for cp in tail_copies():
                cp.start()

    def consume(k, slot):
        b = w * BPW + k
        in_range = k < BPW

        @pl.when(in_range & (b < NBLK))
        def _():
            node_copy(b, slot).wait()
            for cp in id_copies(b, slot):
                cp.wait()
            for g in range(GPB):
                pltpu.sync_copy(
                    nbuf.at[slot, pl.ds(g * GROUP, GROUP)],
                    acc.at[ibuf.at[slot, g]], add=True)

        @pl.when(in_range & (b == TAILBLK))
        def _():
            for cp in tail_copies():
                cp.wait()
            pltpu.sync_copy(
                tnbuf.at[pl.ds(0, GROUP)], acc.at[tidx_a], add=True)
            pltpu.sync_copy(
                tnbuf.at[pl.ds(GROUP, TAIL_REM)], acc.at[tidx_b], add=True)

    start_load(0, 0)
    start_load(1, 1)

    def pair(i, carry):
        k = i * 2
        consume(k, 0)
        start_load(k + 2, 0)
        consume(k + 1, 1)
        start_load(k + 3, 1)
        return carry

    lax.fori_loop(0, NPAIR, pair, 0)

    plsc.subcore_barrier()

    seg0 = s * SEGS_PER_TILE
    pltpu.sync_copy(acc.at[pl.ds(seg0, SEGS_PER_TILE)], outbuf)
    pltpu.sync_copy(outbuf, part_ref.at[c, pl.ds(seg0, SEGS_PER_TILE)])


def _combine_body(p_ref, o_ref):
    o_ref[...] = (p_ref[0] + p_ref[1]) * SCALE


def kernel(nodes, segment_ids, num_segments):
    ids = segment_ids.astype(jnp.int32)
    partials = _sc_segment_sum(nodes, ids)
    return pl.pallas_call(
        _combine_body,
        out_shape=jax.ShapeDtypeStruct((G, D), jnp.float32),
    )(partials)


# speedup vs baseline: 2.5809x; 1.0018x over previous
import functools

import jax
import jax.numpy as jnp
from jax import lax
from jax.experimental import pallas as pl
from jax.experimental.pallas import tpu as pltpu
from jax.experimental.pallas import tpu_sc as plsc

N = 100000
D = 128
G = 64
SCALE = float(1562.5) ** (-0.5)

NC = 2
NS = 16
NW = NC * NS
L = 16
GROUP = 128
BLOCK = 256
GPB = BLOCK // GROUP
NBLK = N // BLOCK
TAILBLK = NBLK
TAIL_ROWS = N - NBLK * BLOCK
TAIL_REM = TAIL_ROWS - GROUP
BPW = -(-(NBLK + 1) // NW)
NPAIR = (BPW + 1) // 2
SEGS_PER_TILE = G // NS


@functools.partial(
    pl.kernel,
    out_type=jax.ShapeDtypeStruct((NC, G, D), jnp.float32),
    mesh=plsc.VectorSubcoreMesh(core_axis_name="c", subcore_axis_name="s"),
    compiler_params=pltpu.CompilerParams(use_tc_tiling_on_sc=False),
    scratch_types=[
        pltpu.VMEM((2, BLOCK, D), jnp.float32),
        pltpu.VMEM((2, GPB, GROUP), jnp.int32),
        pltpu.VMEM((TAIL_ROWS, D), jnp.float32),
        pltpu.VMEM((GROUP,), jnp.int32),
        pltpu.VMEM((TAIL_REM,), jnp.int32),
        pltpu.VMEM((SEGS_PER_TILE, D), jnp.float32),
        pltpu.VMEM_SHARED((G, D), jnp.float32),
        pltpu.SemaphoreType.DMA,
        pltpu.SemaphoreType.DMA,
        pltpu.SemaphoreType.DMA,
        pltpu.SemaphoreType.DMA,
        pltpu.SemaphoreType.DMA,
        pltpu.SemaphoreType.DMA,
    ],
)
def _sc_segment_sum(nodes_ref, ids_ref, part_ref,
                    nbuf, ibuf, tnbuf, tidx_a, tidx_b, outbuf, acc,
                    nsem0, nsem1, isem0, isem1, tnsem, tisem):
    c = lax.axis_index("c")
    s = lax.axis_index("s")
    w = s * NC + c
    nsems = (nsem0, nsem1)
    isems = (isem0, isem1)

    zvec = jnp.zeros((L,), jnp.float32)
    for i in range(SEGS_PER_TILE):
        for l in range(D // L):
            outbuf[i, pl.ds(l * L, L)] = zvec
    pltpu.sync_copy(outbuf, acc.at[pl.ds(s * SEGS_PER_TILE, SEGS_PER_TILE)])
    plsc.subcore_barrier()

    def node_copy(b, slot):
        return pltpu.make_async_copy(
            nodes_ref.at[pl.ds(b * BLOCK, BLOCK)], nbuf.at[slot], nsems[slot])

    def id_copies(b, slot):
        return tuple(
            pltpu.make_async_copy(
                ids_ref.at[pl.ds(b * BLOCK + g * GROUP, GROUP)],
                ibuf.at[slot, g], isems[slot])
            for g in range(GPB))

    def tail_copies():
        r0 = NBLK * BLOCK
        return (
            pltpu.make_async_copy(
                nodes_ref.at[pl.ds(r0, TAIL_ROWS)], tnbuf, tnsem),
            pltpu.make_async_copy(
                ids_ref.at[pl.ds(r0, GROUP)], tidx_a, tisem),
            pltpu.make_async_copy(
                ids_ref.at[pl.ds(r0 + GROUP, TAIL_REM)], tidx_b, tisem),
        )

    def start_load(k, slot):
        b = w * BPW + k
        in_range = k < BPW

        @pl.when(in_range & (b < NBLK))
        def _():
            node_copy(b, slot).start()
            for cp in id_copies(b, slot):
                cp.start()

    def consume(k, slot):
        b = w * BPW + k
        in_range = k < BPW

        @pl.when(in_range & (b < NBLK))
        def _():
            node_copy(b, slot).wait()
            for cp in id_copies(b, slot):
                cp.wait()
            for g in range(GPB):
                pltpu.sync_copy(
                    nbuf.at[slot, pl.ds(g * GROUP, GROUP)],
                    acc.at[ibuf.at[slot, g]], add=True)

    @pl.when(w == TAILBLK // BPW)
    def _tail_start():
        for cp in tail_copies():
            cp.start()

    start_load(0, 0)
    start_load(1, 1)

    def pair(i, carry):
        k = i * 2
        consume(k, 0)
        start_load(k + 2, 0)
        consume(k + 1, 1)
        start_load(k + 3, 1)
        return carry

    lax.fori_loop(0, NPAIR, pair, 0)

    @pl.when(w == TAILBLK // BPW)
    def _tail_consume():
        for cp in tail_copies():
            cp.wait()
        pltpu.sync_copy(
            tnbuf.at[pl.ds(0, GROUP)], acc.at[tidx_a], add=True)
        pltpu.sync_copy(
            tnbuf.at[pl.ds(GROUP, TAIL_REM)], acc.at[tidx_b], add=True)

    plsc.subcore_barrier()

    seg0 = s * SEGS_PER_TILE
    pltpu.sync_copy(acc.at[pl.ds(seg0, SEGS_PER_TILE)], outbuf)
    pltpu.sync_copy(outbuf, part_ref.at[c, pl.ds(seg0, SEGS_PER_TILE)])


def _combine_body(p_ref, o_ref):
    o_ref[...] = (p_ref[0] + p_ref[1]) * SCALE


def kernel(nodes, segment_ids, num_segments):
    ids = segment_ids.astype(jnp.int32)
    partials = _sc_segment_sum(nodes, ids)
    return pl.pallas_call(
        _combine_body,
        out_shape=jax.ShapeDtypeStruct((G, D), jnp.float32),
    )(partials)


# speedup vs baseline: 2.6065x; 1.0099x over previous
import functools

import jax
import jax.numpy as jnp
from jax import lax
from jax.experimental import pallas as pl
from jax.experimental.pallas import tpu as pltpu
from jax.experimental.pallas import tpu_sc as plsc

N = 100000
D = 128
G = 64
SCALE = float(1562.5) ** (-0.5)

NC = 2
NS = 16
NW = NC * NS
L = 16
GROUP = 128
BLOCK = 256
GPB = BLOCK // GROUP
NBLK = N // BLOCK
TAILBLK = NBLK
TAIL_ROWS = N - NBLK * BLOCK
TAIL_REM = TAIL_ROWS - GROUP
BPW = -(-(NBLK + 1) // NW)
NPAIR = (BPW + 1) // 2
SEGS_PER_TILE = G // NS


@functools.partial(
    pl.kernel,
    out_type=jax.ShapeDtypeStruct((NC, G, D), jnp.float32),
    mesh=plsc.VectorSubcoreMesh(core_axis_name="c", subcore_axis_name="s"),
    compiler_params=pltpu.CompilerParams(use_tc_tiling_on_sc=False),
    scratch_types=[
        pltpu.VMEM((2, BLOCK, D), jnp.float32),
        pltpu.VMEM((2, GPB, GROUP), jnp.int32),
        pltpu.VMEM((TAIL_ROWS, D), jnp.float32),
        pltpu.VMEM((GROUP,), jnp.int32),
        pltpu.VMEM((TAIL_REM,), jnp.int32),
        pltpu.VMEM((SEGS_PER_TILE, D), jnp.float32),
        pltpu.VMEM_SHARED((G, D), jnp.float32),
        pltpu.SemaphoreType.DMA((2,)),
        pltpu.SemaphoreType.DMA((2,)),
        pltpu.SemaphoreType.DMA,
        pltpu.SemaphoreType.DMA,
    ],
)
def _sc_segment_sum(nodes_ref, ids_ref, part_ref,
                    nbuf, ibuf, tnbuf, tidx_a, tidx_b, outbuf, acc,
                    nsem, isem, tnsem, tisem):
    c = lax.axis_index("c")
    s = lax.axis_index("s")
    w = s * NC + c

    zvec = jnp.zeros((L,), jnp.float32)
    for i in range(SEGS_PER_TILE):
        for l in range(D // L):
            outbuf[i, pl.ds(l * L, L)] = zvec
    pltpu.sync_copy(outbuf, acc.at[pl.ds(s * SEGS_PER_TILE, SEGS_PER_TILE)])
    plsc.subcore_barrier()

    def node_copy(b, slot):
        return pltpu.make_async_copy(
            nodes_ref.at[pl.ds(b * BLOCK, BLOCK)], nbuf.at[slot],
            nsem.at[slot])

    def id_copies(b, slot):
        return tuple(
            pltpu.make_async_copy(
                ids_ref.at[pl.ds(b * BLOCK + g * GROUP, GROUP)],
                ibuf.at[slot, g], isem.at[slot])
            for g in range(GPB))

    def tail_copies():
        r0 = NBLK * BLOCK
        return (
            pltpu.make_async_copy(
                nodes_ref.at[pl.ds(r0, TAIL_ROWS)], tnbuf, tnsem),
            pltpu.make_async_copy(
                ids_ref.at[pl.ds(r0, GROUP)], tidx_a, tisem),
            pltpu.make_async_copy(
                ids_ref.at[pl.ds(r0 + GROUP, TAIL_REM)], tidx_b, tisem),
        )

    def start_load(k, slot):
        b = w * BPW + k
        in_range = k < BPW

        @pl.when(in_range & (b < NBLK))
        def _():
            node_copy(b, slot).start()
            for cp in id_copies(b, slot):
                cp.start()

    def consume(k, slot):
        b = w * BPW + k
        in_range = k < BPW

        @pl.when(in_range & (b < NBLK))
        def _():
            node_copy(b, slot).wait()
            for cp in id_copies(b, slot):
                cp.wait()
            for g in range(GPB):
                pltpu.sync_copy(
                    nbuf.at[slot, pl.ds(g * GROUP, GROUP)],
                    acc.at[ibuf.at[slot, g]], add=True)

    @pl.when(w == TAILBLK // BPW)
    def _tail_start():
        for cp in tail_copies():
            cp.start()

    start_load(0, 0)
    start_load(1, 1)

    def step(k, carry):
        slot = lax.rem(k, 2)
        consume(k, slot)
        start_load(k + 2, slot)
        return carry

    lax.fori_loop(0, BPW, step, 0)

    @pl.when(w == TAILBLK // BPW)
    def _tail_consume():
        for cp in tail_copies():
            cp.wait()
        pltpu.sync_copy(
            tnbuf.at[pl.ds(0, GROUP)], acc.at[tidx_a], add=True)
        pltpu.sync_copy(
            tnbuf.at[pl.ds(GROUP, TAIL_REM)], acc.at[tidx_b], add=True)

    plsc.subcore_barrier()

    seg0 = s * SEGS_PER_TILE
    pltpu.sync_copy(acc.at[pl.ds(seg0, SEGS_PER_TILE)], outbuf)
    pltpu.sync_copy(outbuf, part_ref.at[c, pl.ds(seg0, SEGS_PER_TILE)])


def _combine_body(p_ref, o_ref):
    o_ref[...] = (p_ref[0] + p_ref[1]) * SCALE


def kernel(nodes, segment_ids, num_segments):
    ids = segment_ids.astype(jnp.int32)
    partials = _sc_segment_sum(nodes, ids)
    return pl.pallas_call(
        _combine_body,
        out_shape=jax.ShapeDtypeStruct((G, D), jnp.float32),
    )(partials)


# speedup vs baseline: 2.6703x; 1.0245x over previous
import functools

import jax
import jax.numpy as jnp
from jax import lax
from jax.experimental import pallas as pl
from jax.experimental.pallas import tpu as pltpu
from jax.experimental.pallas import tpu_sc as plsc

N = 100000
D = 128
G = 64
SCALE = float(1562.5) ** (-0.5)

NC = 2
NS = 16
NW = NC * NS
L = 16
GROUP = 128
BLOCK = 256
GPB = BLOCK // GROUP
NBLK = N // BLOCK
TAILBLK = NBLK
TAIL_ROWS = N - NBLK * BLOCK
TAIL_REM = TAIL_ROWS - GROUP
BPW = -(-(NBLK + 1) // NW)
NSLOT = 3
SEGS_PER_TILE = G // NS


@functools.partial(
    pl.kernel,
    out_type=jax.ShapeDtypeStruct((NC, G, D), jnp.float32),
    mesh=plsc.VectorSubcoreMesh(core_axis_name="c", subcore_axis_name="s"),
    compiler_params=pltpu.CompilerParams(use_tc_tiling_on_sc=False),
    scratch_types=[
        pltpu.VMEM((NSLOT, BLOCK, D), jnp.float32),
        pltpu.VMEM((NSLOT, GPB, GROUP), jnp.int32),
        pltpu.VMEM((TAIL_ROWS, D), jnp.float32),
        pltpu.VMEM((GROUP,), jnp.int32),
        pltpu.VMEM((TAIL_REM,), jnp.int32),
        pltpu.VMEM((SEGS_PER_TILE, D), jnp.float32),
        pltpu.VMEM_SHARED((G, D), jnp.float32),
        pltpu.SemaphoreType.DMA((NSLOT,)),
        pltpu.SemaphoreType.DMA((NSLOT,)),
        pltpu.SemaphoreType.DMA,
        pltpu.SemaphoreType.DMA,
    ],
)
def _sc_segment_sum(nodes_ref, ids_ref, part_ref,
                    nbuf, ibuf, tnbuf, tidx_a, tidx_b, outbuf, acc,
                    nsem, isem, tnsem, tisem):
    c = lax.axis_index("c")
    s = lax.axis_index("s")
    w = s * NC + c

    zvec = jnp.zeros((L,), jnp.float32)
    for i in range(SEGS_PER_TILE):
        for l in range(D // L):
            outbuf[i, pl.ds(l * L, L)] = zvec
    pltpu.sync_copy(outbuf, acc.at[pl.ds(s * SEGS_PER_TILE, SEGS_PER_TILE)])
    plsc.subcore_barrier()

    def node_copy(b, slot):
        return pltpu.make_async_copy(
            nodes_ref.at[pl.ds(b * BLOCK, BLOCK)], nbuf.at[slot],
            nsem.at[slot])

    def id_copies(b, slot):
        return tuple(
            pltpu.make_async_copy(
                ids_ref.at[pl.ds(b * BLOCK + g * GROUP, GROUP)],
                ibuf.at[slot, g], isem.at[slot])
            for g in range(GPB))

    def tail_copies():
        r0 = NBLK * BLOCK
        return (
            pltpu.make_async_copy(
                nodes_ref.at[pl.ds(r0, TAIL_ROWS)], tnbuf, tnsem),
            pltpu.make_async_copy(
                ids_ref.at[pl.ds(r0, GROUP)], tidx_a, tisem),
            pltpu.make_async_copy(
                ids_ref.at[pl.ds(r0 + GROUP, TAIL_REM)], tidx_b, tisem),
        )

    def start_load(k, slot):
        b = w * BPW + k
        in_range = k < BPW

        @pl.when(in_range & (b < NBLK))
        def _():
            node_copy(b, slot).start()
            for cp in id_copies(b, slot):
                cp.start()

    def consume(k, slot):
        b = w * BPW + k
        in_range = k < BPW

        @pl.when(in_range & (b < NBLK))
        def _():
            node_copy(b, slot).wait()
            for cp in id_copies(b, slot):
                cp.wait()
            for g in range(GPB):
                pltpu.sync_copy(
                    nbuf.at[slot, pl.ds(g * GROUP, GROUP)],
                    acc.at[ibuf.at[slot, g]], add=True)

    @pl.when(w == TAILBLK // BPW)
    def _tail_start():
        for cp in tail_copies():
            cp.start()

    for p in range(NSLOT - 1):
        start_load(p, p)

    def step(k, carry):
        slot = lax.rem(k, NSLOT)
        start_load(k + NSLOT - 1, lax.rem(k + NSLOT - 1, NSLOT))
        consume(k, slot)
        return carry

    lax.fori_loop(0, BPW, step, 0)

    @pl.when(w == TAILBLK // BPW)
    def _tail_consume():
        for cp in tail_copies():
            cp.wait()
        pltpu.sync_copy(
            tnbuf.at[pl.ds(0, GROUP)], acc.at[tidx_a], add=True)
        pltpu.sync_copy(
            tnbuf.at[pl.ds(GROUP, TAIL_REM)], acc.at[tidx_b], add=True)

    plsc.subcore_barrier()

    seg0 = s * SEGS_PER_TILE
    pltpu.sync_copy(acc.at[pl.ds(seg0, SEGS_PER_TILE)], outbuf)
    pltpu.sync_copy(outbuf, part_ref.at[c, pl.ds(seg0, SEGS_PER_TILE)])


def _combine_body(p_ref, o_ref):
    o_ref[...] = (p_ref[0] + p_ref[1]) * SCALE


def kernel(nodes, segment_ids, num_segments):
    ids = segment_ids.astype(jnp.int32)
    partials = _sc_segment_sum(nodes, ids)
    return pl.pallas_call(
        _combine_body,
        out_shape=jax.ShapeDtypeStruct((G, D), jnp.float32),
    )(partials)
